# baseline (speedup 1.0000x reference)
import dataclasses
import functools

import jax
import jax.numpy as jnp
from jax import lax
from jax.experimental import pallas as pl
from jax.experimental.pallas import tpu as pltpu
from jax.experimental.pallas import tpu_sc as plsc

N_NODES = 10000
N_EDGES = 320000
HIDDEN = 16

NC = 2
NS = 16
NW = NC * NS
CHUNK = 128
CPW = 80
PIPE = 8
PAD_CHUNKS = NW * CPW
PAD_E = PAD_CHUNKS * CHUNK
ROWS_PER_S = 624
TAIL_ROW0 = ROWS_PER_S * NS
TAIL_ROWS = N_NODES - TAIL_ROW0

_MESH = plsc.VectorSubcoreMesh(core_axis_name="c", subcore_axis_name="s",
                               num_cores=NC, num_subcores=NS)

_SC_PARAMS = pltpu.CompilerParams(use_tc_tiling_on_sc=False)
if "needs_layout_passes" in pltpu.CompilerParams.__dataclass_fields__:
    _SC_PARAMS = dataclasses.replace(_SC_PARAMS, needs_layout_passes=False)


def _make_sc_spmm(width):
    with_deg = width == 32

    @functools.partial(
        pl.kernel,
        out_type=jax.ShapeDtypeStruct((NC, N_NODES, width), jnp.float32),
        mesh=_MESH,
        scratch_types=[
            pltpu.VMEM((CPW, 2, CHUNK), jnp.int32),
            pltpu.VMEM((CPW * CHUNK,), jnp.float32),
            *[pltpu.VMEM((CHUNK, HIDDEN), jnp.float32)
              for _ in range(PIPE)],
            pltpu.VMEM((CHUNK, width), jnp.float32),
            pltpu.VMEM((CHUNK, width), jnp.float32),
            pltpu.VMEM_SHARED((N_NODES, width), jnp.float32),
            *[pltpu.SemaphoreType.DMA for _ in range(PIPE)],
            pltpu.SemaphoreType.DMA,
            pltpu.SemaphoreType.DMA,
        ],
        compiler_params=_SC_PARAMS,
    )
    def sc_spmm(h_hbm, ei_hbm, a_hbm, zeros_hbm, out_hbm, edb, av, *rest):
        rows = rest[0:PIPE]
        msg = rest[PIPE:PIPE + 2]
        acc_sh = rest[PIPE + 2]
        sem_g = rest[PIPE + 3:3 + 2 * PIPE]
        sem_s = rest[3 + 2 * PIPE:5 + 2 * PIPE]
        cid = lax.axis_index("c")
        sid = lax.axis_index("s")
        wid = sid * NC + cid
        c0 = wid * CPW

        pltpu.sync_copy(ei_hbm.at[pl.ds(c0, CPW)], edb)
        pltpu.sync_copy(a_hbm.at[pl.ds(c0 * CHUNK, CPW * CHUNK)], av)

        row0 = sid * ROWS_PER_S
        pltpu.sync_copy(zeros_hbm.at[pl.ds(row0, ROWS_PER_S)],
                        acc_sh.at[pl.ds(row0, ROWS_PER_S)])

        @pl.when(sid == NS - 1)
        def _zero_tail():
            pltpu.sync_copy(zeros_hbm.at[pl.ds(TAIL_ROW0, TAIL_ROWS)],
                            acc_sh.at[pl.ds(TAIL_ROW0, TAIL_ROWS)])

        plsc.subcore_barrier()

        for c in range(PIPE - 1):
            pltpu.async_copy(h_hbm.at[edb.at[c, 1]], rows[c], sem_g[c])

        @pl.loop(0, CPW // PIPE)
        def _blk(k):
            for p in range(PIPE):
                c = PIPE * k + p
                pp = p % 2

                @pl.when(c + PIPE - 1 < CPW)
                def _fire_next_gather():
                    pltpu.async_copy(h_hbm.at[edb.at[c + PIPE - 1, 1]],
                                     rows[(p + PIPE - 1) % PIPE],
                                     sem_g[(p + PIPE - 1) % PIPE])

                pltpu.make_async_copy(h_hbm.at[pl.ds(0, CHUNK)],
                                      rows[p], sem_g[p]).wait()

                @pl.when(c >= 2)
                def _drain_scatter():
                    pltpu.make_async_copy(zeros_hbm.at[pl.ds(0, CHUNK)],
                                          msg[pp], sem_s[pp]).wait()

                abase = c * CHUNK

                @pl.loop(0, CHUNK // 16)
                def _grp(g):
                    e0 = g * 16
                    aa = av[pl.ds(abase + e0, 16)]
                    for j in range(16):
                        asplat = jnp.broadcast_to(aa[j], (16,))
                        row = rows[p][e0 + j, 0:16]
                        msg[pp][e0 + j, 0:16] = row * asplat
                        if with_deg:
                            msg[pp][e0 + j, 16:32] = asplat

                pltpu.async_copy(msg[pp], acc_sh.at[edb.at[c, 0]],
                                 sem_s[pp], add=True)

        for pp in (0, 1):
            pltpu.make_async_copy(zeros_hbm.at[pl.ds(0, CHUNK)],
                                  msg[pp], sem_s[pp]).wait()

        plsc.subcore_barrier()
        pltpu.sync_copy(acc_sh.at[pl.ds(row0, ROWS_PER_S)],
                        out_hbm.at[cid, pl.ds(row0, ROWS_PER_S)])

        @pl.when(sid == NS - 1)
        def _drain_tail():
            pltpu.sync_copy(acc_sh.at[pl.ds(TAIL_ROW0, TAIL_ROWS)],
                            out_hbm.at[cid, pl.ds(TAIL_ROW0, TAIL_ROWS)])

    return sc_spmm


_sc_spmm32 = _make_sc_spmm(32)
_sc_spmm16 = _make_sc_spmm(16)


def _tc_in_proj(x, w1t, b1):
    def body(x_ref, w_ref, b_ref, o_ref):
        o_ref[...] = jnp.dot(x_ref[...], w_ref[...],
                             preferred_element_type=jnp.float32) + b_ref[...]
    return pl.pallas_call(
        body,
        out_shape=jax.ShapeDtypeStruct((N_NODES, HIDDEN), jnp.float32),
    )(x, w1t, b1)


def _tc_relu_sum(o1):
    def body(o1_ref, r_ref):
        s = o1_ref[0, :, 0:HIDDEN] + o1_ref[1, :, 0:HIDDEN]
        r_ref[...] = jnp.maximum(s, 0.0)
    return pl.pallas_call(
        body,
        out_shape=jax.ShapeDtypeStruct((N_NODES, HIDDEN), jnp.float32),
    )(o1)


def _tc_out_proj(o2, o1, w2t, b2):
    def body(o2_ref, o1_ref, w_ref, b_ref, out_ref):
        s2 = o2_ref[0] + o2_ref[1]
        deg = o1_ref[0, :, HIDDEN:HIDDEN + 1] + o1_ref[1, :, HIDDEN:HIDDEN + 1]
        out_ref[...] = (jnp.dot(s2, w_ref[...],
                                preferred_element_type=jnp.float32)
                        + deg * b_ref[...])
    return pl.pallas_call(
        body,
        out_shape=jax.ShapeDtypeStruct((N_NODES, 128), jnp.float32),
    )(o2, o1, w2t, b2)


def kernel(X, edge_index, A_vals, W1_w, W1_b, W2_w, W2_b):
    pad = PAD_E - N_EDGES
    pad_ids = jnp.arange(pad, dtype=jnp.int32) % N_NODES
    ei = jnp.concatenate(
        [edge_index.astype(jnp.int32), jnp.stack([pad_ids, pad_ids])], axis=1)
    ei = ei.reshape(2, PAD_CHUNKS, CHUNK).transpose(1, 0, 2)
    A_vals = jnp.pad(A_vals, (0, pad))
    w1t = W1_w.T
    b1 = W1_b[None, :]
    w2t = W2_w.T
    b2 = W2_b[None, :]
    zeros32 = jnp.zeros((N_NODES, 32), jnp.float32)
    zeros16 = jnp.zeros((N_NODES, HIDDEN), jnp.float32)

    h = _tc_in_proj(X, w1t, b1)
    o1 = _sc_spmm32(h, ei, A_vals, zeros32)
    r = _tc_relu_sum(o1)
    o2 = _sc_spmm16(r, ei, A_vals, zeros16)
    return _tc_out_proj(o2, o1, w2t, b2)


# speedup vs baseline: 1.2264x; 1.2264x over previous
import dataclasses
import functools

import jax
import jax.numpy as jnp
from jax import lax
from jax.experimental import pallas as pl
from jax.experimental.pallas import tpu as pltpu
from jax.experimental.pallas import tpu_sc as plsc

N_NODES = 10000
N_EDGES = 320000
HIDDEN = 16

NC = 2
NS = 16
NW = NC * NS
CHUNK = 128
CPW = 80
PIPE = 8
PAD_CHUNKS = NW * CPW
PAD_E = PAD_CHUNKS * CHUNK
ROWS_PER_S = 624
TAIL_ROW0 = ROWS_PER_S * NS
TAIL_ROWS = N_NODES - TAIL_ROW0

_MESH = plsc.VectorSubcoreMesh(core_axis_name="c", subcore_axis_name="s",
                               num_cores=NC, num_subcores=NS)

_SC_PARAMS = pltpu.CompilerParams(use_tc_tiling_on_sc=False)
if "needs_layout_passes" in pltpu.CompilerParams.__dataclass_fields__:
    _SC_PARAMS = dataclasses.replace(_SC_PARAMS, needs_layout_passes=False)


def _make_sc_spmm(width):
    with_deg = width == 32

    @functools.partial(
        pl.kernel,
        out_type=jax.ShapeDtypeStruct((NC, N_NODES, width), jnp.float32),
        mesh=_MESH,
        scratch_types=[
            pltpu.VMEM((CPW, 2, CHUNK), jnp.int32),
            pltpu.VMEM((CPW * CHUNK,), jnp.float32),
            *[pltpu.VMEM((CHUNK, HIDDEN), jnp.float32)
              for _ in range(PIPE)],
            pltpu.VMEM((CHUNK, width), jnp.float32),
            pltpu.VMEM((CHUNK, width), jnp.float32),
            pltpu.VMEM_SHARED((N_NODES, width), jnp.float32),
            *[pltpu.SemaphoreType.DMA for _ in range(PIPE)],
            pltpu.SemaphoreType.DMA,
            pltpu.SemaphoreType.DMA,
        ],
        compiler_params=_SC_PARAMS,
    )
    def sc_spmm(h_hbm, ei_hbm, a_hbm, zeros_hbm, out_hbm, edb, av, *rest):
        rows = rest[0:PIPE]
        msg = rest[PIPE:PIPE + 2]
        acc_sh = rest[PIPE + 2]
        sem_g = rest[PIPE + 3:3 + 2 * PIPE]
        sem_s = rest[3 + 2 * PIPE:5 + 2 * PIPE]
        cid = lax.axis_index("c")
        sid = lax.axis_index("s")
        wid = sid * NC + cid
        c0 = wid * CPW

        pltpu.sync_copy(ei_hbm.at[pl.ds(c0, CPW)], edb)
        pltpu.sync_copy(a_hbm.at[pl.ds(c0 * CHUNK, CPW * CHUNK)], av)

        row0 = sid * ROWS_PER_S
        pltpu.sync_copy(zeros_hbm.at[pl.ds(row0, ROWS_PER_S)],
                        acc_sh.at[pl.ds(row0, ROWS_PER_S)])

        @pl.when(sid == NS - 1)
        def _zero_tail():
            pltpu.sync_copy(zeros_hbm.at[pl.ds(TAIL_ROW0, TAIL_ROWS)],
                            acc_sh.at[pl.ds(TAIL_ROW0, TAIL_ROWS)])

        plsc.subcore_barrier()

        for c in range(PIPE - 1):
            pltpu.async_copy(h_hbm.at[edb.at[c, 1]], rows[c], sem_g[c])

        @pl.loop(0, CPW // PIPE)
        def _blk(k):
            for p in range(PIPE):
                c = PIPE * k + p
                pp = p % 2

                @pl.when(c + PIPE - 1 < CPW)
                def _fire_next_gather():
                    pltpu.async_copy(h_hbm.at[edb.at[c + PIPE - 1, 1]],
                                     rows[(p + PIPE - 1) % PIPE],
                                     sem_g[(p + PIPE - 1) % PIPE])

                pltpu.make_async_copy(h_hbm.at[pl.ds(0, CHUNK)],
                                      rows[p], sem_g[p]).wait()

                @pl.when(c >= 2)
                def _drain_scatter():
                    pltpu.make_async_copy(zeros_hbm.at[pl.ds(0, CHUNK)],
                                          msg[pp], sem_s[pp]).wait()

                abase = c * CHUNK

                @pl.loop(0, CHUNK // 16)
                def _grp(g):
                    e0 = g * 16
                    aa = av[pl.ds(abase + e0, 16)]
                    splats = [jnp.broadcast_to(aa[j], (16,))
                              for j in range(16)]
                    vals = [rows[p][e0 + j, 0:16] * splats[j]
                            for j in range(16)]
                    for j in range(16):
                        msg[pp][e0 + j, 0:16] = vals[j]
                    if with_deg:
                        for j in range(16):
                            msg[pp][e0 + j, 16:32] = splats[j]

                pltpu.async_copy(msg[pp], acc_sh.at[edb.at[c, 0]],
                                 sem_s[pp], add=True)

        for pp in (0, 1):
            pltpu.make_async_copy(zeros_hbm.at[pl.ds(0, CHUNK)],
                                  msg[pp], sem_s[pp]).wait()

        plsc.subcore_barrier()
        pltpu.sync_copy(acc_sh.at[pl.ds(row0, ROWS_PER_S)],
                        out_hbm.at[cid, pl.ds(row0, ROWS_PER_S)])

        @pl.when(sid == NS - 1)
        def _drain_tail():
            pltpu.sync_copy(acc_sh.at[pl.ds(TAIL_ROW0, TAIL_ROWS)],
                            out_hbm.at[cid, pl.ds(TAIL_ROW0, TAIL_ROWS)])

    return sc_spmm


_sc_spmm32 = _make_sc_spmm(32)
_sc_spmm16 = _make_sc_spmm(16)


def _tc_in_proj(x, w1t, b1):
    def body(x_ref, w_ref, b_ref, o_ref):
        o_ref[...] = jnp.dot(x_ref[...], w_ref[...],
                             preferred_element_type=jnp.float32) + b_ref[...]
    return pl.pallas_call(
        body,
        out_shape=jax.ShapeDtypeStruct((N_NODES, HIDDEN), jnp.float32),
    )(x, w1t, b1)


def _tc_relu_sum(o1):
    def body(o1_ref, r_ref):
        s = o1_ref[0, :, 0:HIDDEN] + o1_ref[1, :, 0:HIDDEN]
        r_ref[...] = jnp.maximum(s, 0.0)
    return pl.pallas_call(
        body,
        out_shape=jax.ShapeDtypeStruct((N_NODES, HIDDEN), jnp.float32),
    )(o1)


def _tc_out_proj(o2, o1, w2t, b2):
    def body(o2_ref, o1_ref, w_ref, b_ref, out_ref):
        s2 = o2_ref[0] + o2_ref[1]
        deg = o1_ref[0, :, HIDDEN:HIDDEN + 1] + o1_ref[1, :, HIDDEN:HIDDEN + 1]
        out_ref[...] = (jnp.dot(s2, w_ref[...],
                                preferred_element_type=jnp.float32)
                        + deg * b_ref[...])
    return pl.pallas_call(
        body,
        out_shape=jax.ShapeDtypeStruct((N_NODES, 128), jnp.float32),
    )(o2, o1, w2t, b2)


def kernel(X, edge_index, A_vals, W1_w, W1_b, W2_w, W2_b):
    pad = PAD_E - N_EDGES
    pad_ids = jnp.arange(pad, dtype=jnp.int32) % N_NODES
    ei = jnp.concatenate(
        [edge_index.astype(jnp.int32), jnp.stack([pad_ids, pad_ids])], axis=1)
    ei = ei.reshape(2, PAD_CHUNKS, CHUNK).transpose(1, 0, 2)
    A_vals = jnp.pad(A_vals, (0, pad))
    w1t = W1_w.T
    b1 = W1_b[None, :]
    w2t = W2_w.T
    b2 = W2_b[None, :]
    zeros32 = jnp.zeros((N_NODES, 32), jnp.float32)
    zeros16 = jnp.zeros((N_NODES, HIDDEN), jnp.float32)

    h = _tc_in_proj(X, w1t, b1)
    o1 = _sc_spmm32(h, ei, A_vals, zeros32)
    r = _tc_relu_sum(o1)
    o2 = _sc_spmm16(r, ei, A_vals, zeros16)
    return _tc_out_proj(o2, o1, w2t, b2)
